# baseline (speedup 1.0000x reference)
import functools

import jax
import jax.numpy as jnp
from jax import lax
from jax.experimental import pallas as pl
from jax.experimental.pallas import tpu as pltpu
from jax.experimental.pallas import tpu_sc as plsc

_L = 16
_NW = 32
_C = 128
_DIMS = (32, 64, 128)
_NB = 3


def kernel(x, weights, table):
    B, F = x.shape
    V, D = table.shape
    n_total = B * F
    per_w = n_total // _NW
    n_chunks = per_w // _C
    chunks_per_f = B // _C
    assert n_total % _NW == 0 and per_w % _C == 0 and B % _C == 0 and D % _L == 0
    assert chunks_per_f & (chunks_per_f - 1) == 0
    assert n_chunks >= 2 * _NB and D == max(_DIMS)

    x_split = x.T.reshape(_NW, n_chunks, 1, _C)
    w_pad = jnp.zeros((_L,), jnp.float32).at[: weights.shape[0]].set(weights)

    mesh = plsc.VectorSubcoreMesh(core_axis_name="c", subcore_axis_name="s")

    @functools.partial(
        pl.kernel,
        mesh=mesh,
        out_type=jax.ShapeDtypeStruct((F, B, D), jnp.float32),
        scratch_types=[
            pltpu.VMEM((n_chunks, 1, _C), jnp.int32),
            pltpu.VMEM((_L,), jnp.float32),
            pltpu.VMEM((_NB, _C, D), jnp.float32),
            pltpu.VMEM((_NB, _C, D), jnp.float32),
            pltpu.SemaphoreType.DMA,
            pltpu.SemaphoreType.DMA,
            pltpu.SemaphoreType.DMA,
            pltpu.SemaphoreType.DMA,
            pltpu.SemaphoreType.DMA,
            pltpu.SemaphoreType.DMA,
        ],
    )
    def run(x_hbm, w_hbm, table_hbm, out_hbm, idx_v, w_v, gbuf, obuf, *sems):
        gsem = sems[:_NB]
        wsem = sems[_NB:]
        wid = lax.axis_index("s") * 2 + lax.axis_index("c")
        chunk0 = wid * n_chunks

        pltpu.sync_copy(w_hbm, w_v)
        pltpu.sync_copy(x_hbm.at[wid], idx_v)

        ones = jnp.ones((_L,), jnp.float32)
        w_vec = w_v[...]
        w_bcast = [w_vec[i] * ones for i in range(len(_DIMS))]
        scales = [
            sum(w_bcast[i] for i, dim in enumerate(_DIMS) if jc * _L < dim)
            for jc in range(D // _L)
        ]

        def gather(j, p):
            return pltpu.async_copy(table_hbm.at[idx_v.at[j, 0]], gbuf.at[p], gsem[p])

        def write(j, p):
            c = chunk0 + j
            f = c // chunks_per_f
            b0 = (c % chunks_per_f) * _C
            return pltpu.async_copy(obuf.at[p], out_hbm.at[f, pl.ds(b0, _C)], wsem[p])

        def scale(p):
            @plsc.parallel_loop(0, _C, 1, unroll=4)
            def _(i):
                for jc in range(D // _L):
                    sl = pl.ds(jc * _L, _L)
                    obuf[p, i, sl] = gbuf[p, i, sl] * scales[jc]

        def wait_g(p):
            pltpu.make_async_copy(table_hbm.at[idx_v.at[0, 0]], gbuf.at[p], gsem[p]).wait()

        def wait_w(p):
            pltpu.make_async_copy(obuf.at[p], out_hbm.at[0, pl.ds(0, _C)], wsem[p]).wait()

        for p in range(_NB):
            gather(p, p)

        def body(t, _):
            j = _NB * t
            for p in range(_NB):
                jq = j + p

                @pl.when(jq < n_chunks)
                def _():
                    wait_g(p)

                    @pl.when(jq >= _NB)
                    def _():
                        wait_w(p)

                    scale(p)

                    @pl.when(jq + _NB < n_chunks)
                    def _():
                        gather(jq + _NB, p)

                    write(jq, p)

            return 0

        lax.fori_loop(0, (n_chunks + _NB - 1) // _NB, body, 0)
        for p in range(_NB):
            wait_w(p)

    out_t = run(x_split, w_pad, table)
    return jnp.transpose(out_t, (1, 0, 2))
